# Initial kernel scaffold; baseline (speedup 1.0000x reference)
import jax
import jax.numpy as jnp
from jax.experimental import pallas as pl


def kernel(x_full, edge_index, u, v, W_in, b_in, W_l, b_l, W_r, We1, be1, We2, be2, Wh1, bh1, Wh2, bh2):
    raise NotImplementedError("write your pallas kernel here")


# speedup vs baseline: 4.2942x; 4.2942x over previous
import functools

import jax
import jax.numpy as jnp
from jax import lax
from jax.experimental import pallas as pl
from jax.experimental.pallas import tpu as pltpu
from jax.experimental.pallas import tpu_sc as plsc

N = 10000
E = 320000
D = 128
L_MAX = 5

NC, NS = 2, 16
NW = NC * NS
EPW = E // NW
CH = 80
NCHUNK = EPW // CH
NP = 10240
RPT = NP // NS
ZR = 128

def _sc_agg_body(h_hbm, src_hbm, dst_hbm, out_hbm, src_v, dst_v, rows_v,
                 zero_v, acc_sh, sem):
    c = lax.axis_index("c")
    s = lax.axis_index("s")

    zvec = jnp.zeros((16,), jnp.float32)

    def zfill(i, carry):
        r = i // (D // 16)
        col = (i % (D // 16)) * 16
        zero_v[r, pl.ds(col, 16)] = zvec
        return carry

    lax.fori_loop(0, ZR * (D // 16), zfill, 0)

    row0 = s * RPT

    def zcopy(j, carry):
        pltpu.sync_copy(zero_v, acc_sh.at[pl.ds(row0 + j * ZR, ZR)])
        return carry

    lax.fori_loop(0, RPT // ZR, zcopy, 0)

    plsc.subcore_barrier()

    ebase = (c * NS + s) * EPW

    def body(i, carry):
        b = ebase + i * CH
        pltpu.sync_copy(src_hbm.at[pl.ds(b, CH)], src_v)
        pltpu.sync_copy(dst_hbm.at[pl.ds(b, CH)], dst_v)
        pltpu.async_copy(h_hbm.at[src_v], rows_v, sem).wait()
        pltpu.sync_copy(rows_v, acc_sh.at[dst_v], add=True)
        return carry

    lax.fori_loop(0, NCHUNK, body, 0)

    plsc.subcore_barrier()
    pltpu.sync_copy(acc_sh.at[pl.ds(row0, RPT)],
                    out_hbm.at[c, pl.ds(row0, RPT)])


def _sc_cnt_body(dst_hbm, out_hbm, dst_v, ones_v, zero_v, acc_sh):
    c = lax.axis_index("c")
    s = lax.axis_index("s")

    ovec = jnp.ones((16,), jnp.float32)
    zvec = jnp.zeros((16,), jnp.float32)

    def ofill(i, carry):
        r = i // (D // 16)
        col = (i % (D // 16)) * 16
        ones_v[r, pl.ds(col, 16)] = ovec
        return carry

    lax.fori_loop(0, CH * (D // 16), ofill, 0)

    def zfill(i, carry):
        r = i // (D // 16)
        col = (i % (D // 16)) * 16
        zero_v[r, pl.ds(col, 16)] = zvec
        return carry

    lax.fori_loop(0, ZR * (D // 16), zfill, 0)

    row0 = s * RPT

    def zcopy(j, carry):
        pltpu.sync_copy(zero_v, acc_sh.at[pl.ds(row0 + j * ZR, ZR)])
        return carry

    lax.fori_loop(0, RPT // ZR, zcopy, 0)

    plsc.subcore_barrier()

    ebase = (c * NS + s) * EPW

    def body(i, carry):
        pltpu.sync_copy(dst_hbm.at[pl.ds(ebase + i * CH, CH)], dst_v)
        pltpu.sync_copy(ones_v, acc_sh.at[dst_v], add=True)
        return carry

    lax.fori_loop(0, NCHUNK, body, 0)

    plsc.subcore_barrier()
    pltpu.sync_copy(acc_sh.at[pl.ds(row0, RPT)],
                    out_hbm.at[c, pl.ds(row0, RPT)])


@functools.cache
def _sc_kernels():
    mesh = plsc.VectorSubcoreMesh(core_axis_name="c", subcore_axis_name="s",
                                  num_cores=NC, num_subcores=NS)
    sc_agg = functools.partial(
        pl.kernel,
        out_type=jax.ShapeDtypeStruct((NC, NP, D), jnp.float32),
        mesh=mesh,
        scratch_types=[
            pltpu.VMEM((CH,), jnp.int32),
            pltpu.VMEM((CH,), jnp.int32),
            pltpu.VMEM((CH, D), jnp.float32),
            pltpu.VMEM((ZR, D), jnp.float32),
            pltpu.VMEM_SHARED((NP, D), jnp.float32),
            pltpu.SemaphoreType.DMA,
        ],
    )(_sc_agg_body)
    sc_cnt = functools.partial(
        pl.kernel,
        out_type=jax.ShapeDtypeStruct((NC, NP, D), jnp.float32),
        mesh=mesh,
        scratch_types=[
            pltpu.VMEM((CH,), jnp.int32),
            pltpu.VMEM((CH, D), jnp.float32),
            pltpu.VMEM((ZR, D), jnp.float32),
            pltpu.VMEM_SHARED((NP, D), jnp.float32),
        ],
    )(_sc_cnt_body)
    return sc_agg, sc_cnt


_BLK = 1000


def _h0_body(x_ref, w_ref, b_ref, o_ref):
    o_ref[...] = (jnp.dot(x_ref[...], w_ref[...],
                          preferred_element_type=jnp.float32) + b_ref[...])


def _h0(x, w, b):
    return pl.pallas_call(
        _h0_body,
        grid=(N // _BLK,),
        in_specs=[
            pl.BlockSpec((_BLK, D), lambda i: (i, 0)),
            pl.BlockSpec((D, D), lambda i: (0, 0)),
            pl.BlockSpec((1, D), lambda i: (0, 0)),
        ],
        out_specs=pl.BlockSpec((_BLK, D), lambda i: (i, 0)),
        out_shape=jax.ShapeDtypeStruct((N, D), jnp.float32),
    )(x, w, b.reshape(1, D))


def _layer_body(ag_ref, cn_ref, h_ref, wl_ref, bl_ref, wr_ref, o_ref):
    agg = ag_ref[0] + ag_ref[1]
    cnt = cn_ref[0][:, 0:1] + cn_ref[1][:, 0:1]
    mean = agg / jnp.maximum(cnt, 1.0)
    o_ref[...] = jnp.maximum(
        jnp.dot(mean, wl_ref[...], preferred_element_type=jnp.float32)
        + jnp.dot(h_ref[...], wr_ref[...], preferred_element_type=jnp.float32)
        + bl_ref[...],
        0.0)


def _layer(agg2, cnt2, h, wl, bl, wr):
    return pl.pallas_call(
        _layer_body,
        grid=(N // _BLK,),
        in_specs=[
            pl.BlockSpec((NC, _BLK, D), lambda i: (0, i, 0)),
            pl.BlockSpec((NC, _BLK, D), lambda i: (0, i, 0)),
            pl.BlockSpec((_BLK, D), lambda i: (i, 0)),
            pl.BlockSpec((D, D), lambda i: (0, 0)),
            pl.BlockSpec((1, D), lambda i: (0, 0)),
            pl.BlockSpec((D, D), lambda i: (0, 0)),
        ],
        out_specs=pl.BlockSpec((_BLK, D), lambda i: (i, 0)),
        out_shape=jax.ShapeDtypeStruct((N, D), jnp.float32),
    )(agg2, cnt2, h, wl, bl.reshape(1, D), wr)


def _heads_body(hu_ref, hv_ref, we1a_ref, we1b_ref, we1c_ref, be1_ref,
                we2_ref, be2_ref, wh1a_ref, wh1b_ref, wh1c_ref, bh1_ref,
                wh2_ref, bh2_ref, fs_ref, ed_ref, al_ref, sc_ref):
    hu = hu_ref[...]
    hv = hv_ref[...]
    e = jnp.maximum(
        jnp.dot(hu, we1a_ref[...], preferred_element_type=jnp.float32)
        + jnp.dot(hv, we1b_ref[...], preferred_element_type=jnp.float32)
        + jnp.dot(hu * hv, we1c_ref[...], preferred_element_type=jnp.float32)
        + be1_ref[...],
        0.0)
    scores = (jnp.dot(e, we2_ref[...], preferred_element_type=jnp.float32)
              + be2_ref[...])
    g = jnp.maximum(
        jnp.dot(hu, wh1a_ref[...], preferred_element_type=jnp.float32)
        + jnp.dot(hv, wh1b_ref[...], preferred_element_type=jnp.float32)
        + scores * wh1c_ref[...]
        + bh1_ref[...],
        0.0)
    z = (jnp.dot(g, wh2_ref[...], preferred_element_type=jnp.float32)
         + bh2_ref[...])
    p = 1.0 / (1.0 + jnp.exp(-z))
    pn = jnp.maximum(1.0 - p, 1e-30)
    tri = jnp.tril(jnp.ones((L_MAX, L_MAX), jnp.float32), -1)
    cum = jnp.dot(tri, jnp.log(pn), preferred_element_type=jnp.float32)
    alpha = p * jnp.exp(cum)
    alpha = alpha / (jnp.sum(alpha) + 1e-8)
    depths = (lax.broadcasted_iota(jnp.int32, (L_MAX, 1), 0) + 1
              ).astype(jnp.float32)
    fs_ref[...] = jnp.sum(alpha * scores).reshape(1, 1)
    ed_ref[...] = jnp.sum(alpha * depths).reshape(1, 1)
    al_ref[...] = alpha
    sc_ref[...] = scores


def _heads(hu, hv, We1, be1, We2, be2, Wh1, bh1, Wh2, bh2):
    return pl.pallas_call(
        _heads_body,
        out_shape=(
            jax.ShapeDtypeStruct((1, 1), jnp.float32),
            jax.ShapeDtypeStruct((1, 1), jnp.float32),
            jax.ShapeDtypeStruct((L_MAX, 1), jnp.float32),
            jax.ShapeDtypeStruct((L_MAX, 1), jnp.float32),
        ),
    )(hu, hv, We1[:D], We1[D:2 * D], We1[2 * D:], be1.reshape(1, D),
      We2, be2.reshape(1, 1), Wh1[:D], Wh1[D:2 * D], Wh1[2 * D:],
      bh1.reshape(1, 64), Wh2, bh2.reshape(1, 1))


def kernel(x_full, edge_index, u, v, W_in, b_in, W_l, b_l, W_r,
           We1, be1, We2, be2, Wh1, bh1, Wh2, bh2):
    src = edge_index[0]
    dst = edge_index[1]

    sc_agg, sc_cnt = _sc_kernels()
    cnt2 = sc_cnt(dst)
    h = _h0(x_full, W_in, b_in)

    hu_list, hv_list = [], []
    for _ in range(L_MAX):
        agg2 = sc_agg(h, src, dst)
        h = _layer(agg2, cnt2, h, W_l, b_l, W_r)
        hu_list.append(lax.dynamic_slice(h, (u, 0), (1, D)))
        hv_list.append(lax.dynamic_slice(h, (v, 0), (1, D)))

    hu = jnp.concatenate(hu_list, axis=0)
    hv = jnp.concatenate(hv_list, axis=0)
    fs, ed, al, _ = _heads(hu, hv, We1, be1, We2, be2, Wh1, bh1, Wh2, bh2)
    return fs.reshape(()), ed.reshape(()), al.reshape(L_MAX)


# speedup vs baseline: 6.2170x; 1.4478x over previous
import functools

import jax
import jax.numpy as jnp
from jax import lax
from jax.experimental import pallas as pl
from jax.experimental.pallas import tpu as pltpu
from jax.experimental.pallas import tpu_sc as plsc

N = 10000
E = 320000
D = 128
L_MAX = 5

NC, NS = 2, 16
NW = NC * NS
EPW = E // NW
CH = 80
NCHUNK = EPW // CH
CHA = 80
NCHF = EPW // CHA
NP = 10240
RPT = NP // NS
ZR = 128
ZRA = 32

def _sc_agg_body(h_hbm, src_hbm, dst_hbm, out_hbm, src_all, dst_all,
                 src_a, dst_a, src_b, dst_b, rows_a, rows_b,
                 zero_v, acc_sh, sem_a, sem_b):
    c = lax.axis_index("c")
    s = lax.axis_index("s")
    ebase = (c * NS + s) * EPW

    pltpu.sync_copy(src_hbm.at[pl.ds(ebase, EPW)], src_all)
    pltpu.sync_copy(dst_hbm.at[pl.ds(ebase, EPW)], dst_all)

    zvec = jnp.zeros((16,), jnp.float32)

    def zfill(i, carry):
        r = i // (D // 16)
        col = (i % (D // 16)) * 16
        zero_v[r, pl.ds(col, 16)] = zvec
        return carry

    lax.fori_loop(0, ZRA * (D // 16), zfill, 0)

    def idx_copy(k, sbuf, dbuf):
        for j in range(CHA // 16):
            sbuf[pl.ds(j * 16, 16)] = src_all[pl.ds(k * CHA + j * 16, 16)]
            dbuf[pl.ds(j * 16, 16)] = dst_all[pl.ds(k * CHA + j * 16, 16)]

    idx_copy(0, src_a, dst_a)
    pltpu.async_copy(h_hbm.at[src_a], rows_a, sem_a)

    row0 = s * RPT

    def zcopy(j, carry):
        pltpu.sync_copy(zero_v, acc_sh.at[pl.ds(row0 + j * ZRA, ZRA)])
        return carry

    lax.fori_loop(0, RPT // ZRA, zcopy, 0)
    plsc.subcore_barrier()

    def wait(rows, sem):
        pltpu.make_async_copy(h_hbm.at[pl.ds(0, CHA)], rows, sem).wait()

    def body(i, carry):
        cb = 2 * i + 1
        idx_copy(cb, src_b, dst_b)
        pltpu.async_copy(h_hbm.at[src_b], rows_b, sem_b)
        wait(rows_a, sem_a)
        pltpu.sync_copy(rows_a, acc_sh.at[dst_a], add=True)
        ca = 2 * i + 2

        @pl.when(ca < NCHF)
        def _():
            idx_copy(ca, src_a, dst_a)
            pltpu.async_copy(h_hbm.at[src_a], rows_a, sem_a)

        wait(rows_b, sem_b)
        pltpu.sync_copy(rows_b, acc_sh.at[dst_b], add=True)
        return carry

    lax.fori_loop(0, NCHF // 2, body, 0)

    wait(rows_a, sem_a)
    pltpu.sync_copy(rows_a, acc_sh.at[dst_a], add=True)

    plsc.subcore_barrier()
    pltpu.sync_copy(acc_sh.at[pl.ds(row0, RPT)],
                    out_hbm.at[c, pl.ds(row0, RPT)])


def _sc_cnt_body(dst_hbm, out_hbm, dst_v, ones_v, zero_v, acc_sh):
    c = lax.axis_index("c")
    s = lax.axis_index("s")

    ovec = jnp.ones((16,), jnp.float32)
    zvec = jnp.zeros((16,), jnp.float32)

    def ofill(i, carry):
        r = i // (D // 16)
        col = (i % (D // 16)) * 16
        ones_v[r, pl.ds(col, 16)] = ovec
        return carry

    lax.fori_loop(0, CH * (D // 16), ofill, 0)

    def zfill(i, carry):
        r = i // (D // 16)
        col = (i % (D // 16)) * 16
        zero_v[r, pl.ds(col, 16)] = zvec
        return carry

    lax.fori_loop(0, ZR * (D // 16), zfill, 0)

    row0 = s * RPT

    def zcopy(j, carry):
        pltpu.sync_copy(zero_v, acc_sh.at[pl.ds(row0 + j * ZR, ZR)])
        return carry

    lax.fori_loop(0, RPT // ZR, zcopy, 0)

    plsc.subcore_barrier()

    ebase = (c * NS + s) * EPW

    def body(i, carry):
        pltpu.sync_copy(dst_hbm.at[pl.ds(ebase + i * CH, CH)], dst_v)
        pltpu.sync_copy(ones_v, acc_sh.at[dst_v], add=True)
        return carry

    lax.fori_loop(0, NCHUNK, body, 0)

    plsc.subcore_barrier()
    pltpu.sync_copy(acc_sh.at[pl.ds(row0, RPT)],
                    out_hbm.at[c, pl.ds(row0, RPT)])


@functools.cache
def _sc_kernels():
    mesh = plsc.VectorSubcoreMesh(core_axis_name="c", subcore_axis_name="s",
                                  num_cores=NC, num_subcores=NS)
    sc_agg = functools.partial(
        pl.kernel,
        out_type=jax.ShapeDtypeStruct((NC, NP, D), jnp.float32),
        mesh=mesh,
        scratch_types=[
            pltpu.VMEM((EPW,), jnp.int32),
            pltpu.VMEM((EPW,), jnp.int32),
            pltpu.VMEM((CHA,), jnp.int32),
            pltpu.VMEM((CHA,), jnp.int32),
            pltpu.VMEM((CHA,), jnp.int32),
            pltpu.VMEM((CHA,), jnp.int32),
            pltpu.VMEM((CHA, D), jnp.float32),
            pltpu.VMEM((CHA, D), jnp.float32),
            pltpu.VMEM((ZRA, D), jnp.float32),
            pltpu.VMEM_SHARED((NP, D), jnp.float32),
            pltpu.SemaphoreType.DMA,
            pltpu.SemaphoreType.DMA,
        ],
    )(_sc_agg_body)
    sc_cnt = functools.partial(
        pl.kernel,
        out_type=jax.ShapeDtypeStruct((NC, NP, D), jnp.float32),
        mesh=mesh,
        scratch_types=[
            pltpu.VMEM((CH,), jnp.int32),
            pltpu.VMEM((CH, D), jnp.float32),
            pltpu.VMEM((ZR, D), jnp.float32),
            pltpu.VMEM_SHARED((NP, D), jnp.float32),
        ],
    )(_sc_cnt_body)
    return sc_agg, sc_cnt


_BLK = 1000


def _h0_body(x_ref, w_ref, b_ref, o_ref):
    o_ref[...] = (jnp.dot(x_ref[...], w_ref[...],
                          preferred_element_type=jnp.float32) + b_ref[...])


def _h0(x, w, b):
    return pl.pallas_call(
        _h0_body,
        grid=(N // _BLK,),
        in_specs=[
            pl.BlockSpec((_BLK, D), lambda i: (i, 0)),
            pl.BlockSpec((D, D), lambda i: (0, 0)),
            pl.BlockSpec((1, D), lambda i: (0, 0)),
        ],
        out_specs=pl.BlockSpec((_BLK, D), lambda i: (i, 0)),
        out_shape=jax.ShapeDtypeStruct((N, D), jnp.float32),
    )(x, w, b.reshape(1, D))


def _layer_body(ag_ref, cn_ref, h_ref, wl_ref, bl_ref, wr_ref, o_ref):
    agg = ag_ref[0] + ag_ref[1]
    cnt = cn_ref[0][:, 0:1] + cn_ref[1][:, 0:1]
    mean = agg / jnp.maximum(cnt, 1.0)
    o_ref[...] = jnp.maximum(
        jnp.dot(mean, wl_ref[...], preferred_element_type=jnp.float32)
        + bl_ref[...]
        + jnp.dot(h_ref[...], wr_ref[...], preferred_element_type=jnp.float32),
        0.0)


def _layer(agg2, cnt2, h, wl, bl, wr):
    return pl.pallas_call(
        _layer_body,
        grid=(N // _BLK,),
        in_specs=[
            pl.BlockSpec((NC, _BLK, D), lambda i: (0, i, 0)),
            pl.BlockSpec((NC, _BLK, D), lambda i: (0, i, 0)),
            pl.BlockSpec((_BLK, D), lambda i: (i, 0)),
            pl.BlockSpec((D, D), lambda i: (0, 0)),
            pl.BlockSpec((1, D), lambda i: (0, 0)),
            pl.BlockSpec((D, D), lambda i: (0, 0)),
        ],
        out_specs=pl.BlockSpec((_BLK, D), lambda i: (i, 0)),
        out_shape=jax.ShapeDtypeStruct((N, D), jnp.float32),
    )(agg2, cnt2, h, wl, bl.reshape(1, D), wr)


def _heads_body(hu_ref, hv_ref, we1a_ref, we1b_ref, we1c_ref, be1_ref,
                we2_ref, be2_ref, wh1a_ref, wh1b_ref, wh1c_ref, bh1_ref,
                wh2_ref, bh2_ref, fs_ref, ed_ref, al_ref, sc_ref):
    hu = hu_ref[...]
    hv = hv_ref[...]
    e = jnp.maximum(
        jnp.dot(hu, we1a_ref[...], preferred_element_type=jnp.float32)
        + jnp.dot(hv, we1b_ref[...], preferred_element_type=jnp.float32)
        + jnp.dot(hu * hv, we1c_ref[...], preferred_element_type=jnp.float32)
        + be1_ref[...],
        0.0)
    scores = (jnp.dot(e, we2_ref[...], preferred_element_type=jnp.float32)
              + be2_ref[...])
    g = jnp.maximum(
        jnp.dot(hu, wh1a_ref[...], preferred_element_type=jnp.float32)
        + jnp.dot(hv, wh1b_ref[...], preferred_element_type=jnp.float32)
        + jnp.dot(scores, wh1c_ref[...], preferred_element_type=jnp.float32)
        + bh1_ref[...],
        0.0)
    z = (jnp.dot(g, wh2_ref[...], preferred_element_type=jnp.float32)
         + bh2_ref[...])
    p = 1.0 / (1.0 + jnp.exp(-z))
    pn = jnp.maximum(1.0 - p, 1e-30)
    tri = jnp.tril(jnp.ones((L_MAX, L_MAX), jnp.float32), -1)
    cum = jnp.dot(tri, jnp.log(pn), preferred_element_type=jnp.float32)
    alpha = p * jnp.exp(cum)
    alpha = alpha / (jnp.sum(alpha) + 1e-8)
    depths = (lax.broadcasted_iota(jnp.int32, (L_MAX, 1), 0) + 1
              ).astype(jnp.float32)
    fs_ref[...] = jnp.sum(alpha * scores).reshape(1, 1)
    ed_ref[...] = jnp.sum(alpha * depths).reshape(1, 1)
    al_ref[...] = alpha
    sc_ref[...] = scores


def _heads(hu, hv, We1, be1, We2, be2, Wh1, bh1, Wh2, bh2):
    return pl.pallas_call(
        _heads_body,
        out_shape=(
            jax.ShapeDtypeStruct((1, 1), jnp.float32),
            jax.ShapeDtypeStruct((1, 1), jnp.float32),
            jax.ShapeDtypeStruct((L_MAX, 1), jnp.float32),
            jax.ShapeDtypeStruct((L_MAX, 1), jnp.float32),
        ),
    )(hu, hv, We1[:D], We1[D:2 * D], We1[2 * D:], be1.reshape(1, D),
      We2, be2.reshape(1, 1), Wh1[:D], Wh1[D:2 * D], Wh1[2 * D:],
      bh1.reshape(1, 64), Wh2, bh2.reshape(1, 1))


def kernel(x_full, edge_index, u, v, W_in, b_in, W_l, b_l, W_r,
           We1, be1, We2, be2, Wh1, bh1, Wh2, bh2):
    order = jnp.argsort(edge_index[1], stable=True)
    src = edge_index[0][order]
    dst = edge_index[1][order]

    sc_agg, sc_cnt = _sc_kernels()
    cnt2 = sc_cnt(dst)
    h = _h0(x_full, W_in, b_in)

    hu_list, hv_list = [], []
    for _ in range(L_MAX):
        agg2 = sc_agg(h, src, dst)
        h = _layer(agg2, cnt2, h, W_l, b_l, W_r)
        hu_list.append(lax.dynamic_slice(h, (u, 0), (1, D)))
        hv_list.append(lax.dynamic_slice(h, (v, 0), (1, D)))

    hu = jnp.concatenate(hu_list, axis=0)
    hv = jnp.concatenate(hv_list, axis=0)
    fs, ed, al, _ = _heads(hu, hv, We1, be1, We2, be2, Wh1, bh1, Wh2, bh2)
    return fs.reshape(()), ed.reshape(()), al.reshape(L_MAX)
